# baseline (speedup 1.0000x reference)
import jax
import jax.numpy as jnp
from jax.experimental import pallas as pl
from jax.experimental.pallas import tpu as pltpu

B, N, D, H = 4, 2048, 128, 128
K = 8
CH = N // K


def _fused_kernel(graph_hbm, embA_ref, embB_ref, w1_ref, b1_ref, w2_ref,
                  b2_ref, ln_a_ref, ln_b_ref, fw1_ref, fb1_ref, fw2_ref,
                  fb2_ref, out_ref, g_full, x1_s, x2_s, sems):
    i = pl.program_id(0)
    c = pl.program_id(1)
    cur = i * K + c

    def chunk_copy(chunk_id, slot):
        bi = chunk_id // K
        ci = chunk_id % K
        return pltpu.make_async_copy(
            graph_hbm.at[bi, pl.ds(ci * CH, CH), :],
            g_full.at[bi % 2, pl.ds(ci * CH, CH), :],
            sems.at[slot])

    @pl.when(i < B)
    def _dma():
        @pl.when(cur == 0)
        def _():
            chunk_copy(0, 0).start()

        @pl.when(cur + 1 < B * K)
        def _():
            chunk_copy(cur + 1, (cur + 1) % 2).start()

    @pl.when(i < B)
    def _phase0():
        @pl.when(c == 0)
        def _():
            x1_s[...] = jnp.dot(embA_ref[0], w1_ref[...],
                                preferred_element_type=jnp.float32
                                ) + b1_ref[...]

        chunk_copy(cur, cur % 2).wait()
        g_chunk = g_full[i % 2, pl.ds(c * CH, CH), :]
        h = jnp.dot(g_chunk, x1_s[...], preferred_element_type=jnp.float32)
        h = jnp.maximum(h, 0.0)
        x2_s[i % 2, pl.ds(c * CH, CH), :] = jnp.dot(
            h, w2_ref[...], preferred_element_type=jnp.float32) + b2_ref[...]

    @pl.when(i > 0)
    def _phase1():
        eps = 1e-6
        g_chunk = g_full[(i - 1) % 2, pl.ds(c * CH, CH), :]
        temp = jnp.dot(g_chunk, x2_s[(i - 1) % 2],
                       preferred_element_type=jnp.float32)
        mean = jnp.mean(temp, axis=-1, keepdims=True)
        cent = temp - mean
        var = jnp.sum(cent * cent, axis=-1, keepdims=True) / (D - 1)
        std = jnp.sqrt(var)
        normed = ln_a_ref[...] * cent / (std + eps) + ln_b_ref[...]
        num_fea = normed + embB_ref[0]
        ff = jnp.dot(num_fea, fw1_ref[...],
                     preferred_element_type=jnp.float32) + fb1_ref[...]
        ff = jnp.maximum(ff, 0.0)
        ff = jnp.dot(ff, fw2_ref[...],
                     preferred_element_type=jnp.float32) + fb2_ref[...]
        out_ref[0] = ff + num_fea


@jax.jit
def kernel(emb, graph, gcn_W1, gcn_b1, gcn_W2, gcn_b2, ln_a, ln_b,
           ff_W1, ff_b1, ff_W2, ff_b2):
    out = pl.pallas_call(
        _fused_kernel,
        grid=(B + 1, K),
        in_specs=[
            pl.BlockSpec(memory_space=pl.ANY),
            pl.BlockSpec((1, N, D),
                         lambda i, c: (jnp.minimum(i, B - 1), 0, 0)),
            pl.BlockSpec((1, CH, D),
                         lambda i, c: (jnp.maximum(i - 1, 0), c, 0)),
            pl.BlockSpec((D, H), lambda i, c: (0, 0)),
            pl.BlockSpec((H,), lambda i, c: (0,)),
            pl.BlockSpec((H, D), lambda i, c: (0, 0)),
            pl.BlockSpec((D,), lambda i, c: (0,)),
            pl.BlockSpec((D,), lambda i, c: (0,)),
            pl.BlockSpec((D,), lambda i, c: (0,)),
            pl.BlockSpec((D, H), lambda i, c: (0, 0)),
            pl.BlockSpec((H,), lambda i, c: (0,)),
            pl.BlockSpec((H, D), lambda i, c: (0, 0)),
            pl.BlockSpec((D,), lambda i, c: (0,)),
        ],
        out_specs=pl.BlockSpec(
            (1, CH, D),
            lambda i, c: (jnp.maximum(i - 1, 0), jnp.where(i > 0, c, 0), 0)),
        out_shape=jax.ShapeDtypeStruct((B, N, D), jnp.float32),
        scratch_shapes=[pltpu.VMEM((2, N, N), jnp.float32),
                        pltpu.VMEM((N, H), jnp.float32),
                        pltpu.VMEM((2, N, D), jnp.float32),
                        pltpu.SemaphoreType.DMA((2,))],
        compiler_params=pltpu.CompilerParams(
            vmem_limit_bytes=110 * 1024 * 1024),
    )(graph, emb, emb, gcn_W1, gcn_b1, gcn_W2, gcn_b2, ln_a, ln_b,
      ff_W1, ff_b1, ff_W2, ff_b2)
    return out


# speedup vs baseline: 1.6616x; 1.6616x over previous
import jax
import jax.numpy as jnp
from jax.experimental import pallas as pl
from jax.experimental.pallas import tpu as pltpu

B, N, D, H = 4, 2048, 128, 128
K = 2
CH = N // K


def _fused_kernel(graph_hbm, embA_ref, embB_ref, w1_ref, b1_ref, w2_ref,
                  b2_ref, ln_a_ref, ln_b_ref, fw1_ref, fb1_ref, fw2_ref,
                  fb2_ref, out_ref, g_full, x1_s, x2_s, sems):
    i = pl.program_id(0)
    c = pl.program_id(1)
    cur = i * K + c

    def chunk_copy(chunk_id, slot):
        bi = chunk_id // K
        ci = chunk_id % K
        return pltpu.make_async_copy(
            graph_hbm.at[bi, pl.ds(ci * CH, CH), :],
            g_full.at[bi % 2, pl.ds(ci * CH, CH), :],
            sems.at[slot])

    @pl.when(i < B)
    def _dma():
        @pl.when(cur == 0)
        def _():
            chunk_copy(0, 0).start()

        @pl.when(cur + 1 < B * K)
        def _():
            chunk_copy(cur + 1, (cur + 1) % 2).start()

    @pl.when(i < B)
    def _phase0():
        @pl.when(c == 0)
        def _():
            x1_s[...] = jnp.dot(embA_ref[0], w1_ref[...],
                                preferred_element_type=jnp.float32
                                ) + b1_ref[...]

        chunk_copy(cur, cur % 2).wait()
        g_chunk = g_full[i % 2, pl.ds(c * CH, CH), :]
        h = jnp.dot(g_chunk, x1_s[...], preferred_element_type=jnp.float32)
        h = jnp.maximum(h, 0.0)
        x2_s[i % 2, pl.ds(c * CH, CH), :] = jnp.dot(
            h, w2_ref[...], preferred_element_type=jnp.float32) + b2_ref[...]

    @pl.when(i > 0)
    def _phase1():
        eps = 1e-6
        g_chunk = g_full[(i - 1) % 2, pl.ds(c * CH, CH), :]
        temp = jnp.dot(g_chunk, x2_s[(i - 1) % 2],
                       preferred_element_type=jnp.float32)
        mean = jnp.mean(temp, axis=-1, keepdims=True)
        cent = temp - mean
        var = jnp.sum(cent * cent, axis=-1, keepdims=True) / (D - 1)
        std = jnp.sqrt(var)
        normed = ln_a_ref[...] * cent / (std + eps) + ln_b_ref[...]
        num_fea = normed + embB_ref[0]
        ff = jnp.dot(num_fea, fw1_ref[...],
                     preferred_element_type=jnp.float32) + fb1_ref[...]
        ff = jnp.maximum(ff, 0.0)
        ff = jnp.dot(ff, fw2_ref[...],
                     preferred_element_type=jnp.float32) + fb2_ref[...]
        out_ref[0] = ff + num_fea


@jax.jit
def kernel(emb, graph, gcn_W1, gcn_b1, gcn_W2, gcn_b2, ln_a, ln_b,
           ff_W1, ff_b1, ff_W2, ff_b2):
    out = pl.pallas_call(
        _fused_kernel,
        grid=(B + 1, K),
        in_specs=[
            pl.BlockSpec(memory_space=pl.ANY),
            pl.BlockSpec((1, N, D),
                         lambda i, c: (jnp.minimum(i, B - 1), 0, 0)),
            pl.BlockSpec((1, CH, D),
                         lambda i, c: (jnp.maximum(i - 1, 0), c, 0)),
            pl.BlockSpec((D, H), lambda i, c: (0, 0)),
            pl.BlockSpec((H,), lambda i, c: (0,)),
            pl.BlockSpec((H, D), lambda i, c: (0, 0)),
            pl.BlockSpec((D,), lambda i, c: (0,)),
            pl.BlockSpec((D,), lambda i, c: (0,)),
            pl.BlockSpec((D,), lambda i, c: (0,)),
            pl.BlockSpec((D, H), lambda i, c: (0, 0)),
            pl.BlockSpec((H,), lambda i, c: (0,)),
            pl.BlockSpec((H, D), lambda i, c: (0, 0)),
            pl.BlockSpec((D,), lambda i, c: (0,)),
        ],
        out_specs=pl.BlockSpec(
            (1, CH, D),
            lambda i, c: (jnp.maximum(i - 1, 0), jnp.where(i > 0, c, 0), 0)),
        out_shape=jax.ShapeDtypeStruct((B, N, D), jnp.float32),
        scratch_shapes=[pltpu.VMEM((2, N, N), jnp.float32),
                        pltpu.VMEM((N, H), jnp.float32),
                        pltpu.VMEM((2, N, D), jnp.float32),
                        pltpu.SemaphoreType.DMA((2,))],
        compiler_params=pltpu.CompilerParams(
            vmem_limit_bytes=110 * 1024 * 1024),
    )(graph, emb, emb, gcn_W1, gcn_b1, gcn_W2, gcn_b2, ln_a, ln_b,
      ff_W1, ff_b1, ff_W2, ff_b2)
    return out
